# Initial kernel scaffold; baseline (speedup 1.0000x reference)
import functools

import jax
import jax.numpy as jnp
from jax import lax
from jax.experimental import pallas as pl
from jax.experimental.pallas import tpu as pltpu
from jax.experimental.pallas import tpu_sc as plsc

_NUM_BLUEPRINTS = 100000
_NULL_IDX = _NUM_BLUEPRINTS
_D = 64
_NC = 2
_NS = 16
_NW = _NC * _NS
_B = 4096 * 26
_CHUNK = 128
_B_PER_W = _B // _NW
_N_CHUNKS = _B_PER_W // _CHUNK


@functools.partial(
    pl.kernel,
    out_type=jax.ShapeDtypeStruct((_B, _D), jnp.float32),
    mesh=plsc.VectorSubcoreMesh(core_axis_name="c", subcore_axis_name="s"),
    scratch_types=[
        pltpu.VMEM((_N_CHUNKS, _CHUNK), jnp.int32),
        pltpu.VMEM((_CHUNK, _D), jnp.float32),
        pltpu.SemaphoreType.DMA,
    ],
)
def _sc_gather(idx_hbm, table_hbm, out_hbm, idx_v, rows_v, sem):
    wid = lax.axis_index("s") * _NC + lax.axis_index("c")
    cbase = wid * _N_CHUNKS

    pltpu.sync_copy(idx_hbm.at[pl.ds(cbase, _N_CHUNKS)], idx_v)

    def _remap(j, carry):
        for i in range(_CHUNK // 16):
            v = idx_v[j, pl.ds(i * 16, 16)]
            u = jnp.minimum(plsc.bitcast(v, jnp.uint32), jnp.uint32(_NULL_IDX))
            idx_v[j, pl.ds(i * 16, 16)] = plsc.bitcast(u, jnp.int32)
        return carry

    lax.fori_loop(0, _N_CHUNKS, _remap, 0)

    def _chunk(j, carry):
        pltpu.async_copy(table_hbm.at[idx_v.at[j]], rows_v, sem).wait()
        pltpu.sync_copy(rows_v, out_hbm.at[pl.ds((cbase + j) * _CHUNK, _CHUNK)])
        return carry

    lax.fori_loop(0, _N_CHUNKS, _chunk, 0)


def kernel(blueprint_indices, table):
    idx2d = blueprint_indices.reshape(_NW * _N_CHUNKS, _CHUNK).astype(jnp.int32)
    out = _sc_gather(idx2d, table)
    return out.reshape(4096, 26, _D)


# speedup vs baseline: 1.0965x; 1.0965x over previous
import functools

import jax
import jax.numpy as jnp
from jax import lax
from jax.experimental import pallas as pl
from jax.experimental.pallas import tpu as pltpu
from jax.experimental.pallas import tpu_sc as plsc

_NUM_BLUEPRINTS = 100000
_NULL_IDX = _NUM_BLUEPRINTS
_D = 64
_NC = 2
_NS = 16
_NW = _NC * _NS
_B = 4096 * 26
_CHUNK = 128
_B_PER_W = _B // _NW
_N_CHUNKS = _B_PER_W // _CHUNK


@functools.partial(
    pl.kernel,
    out_type=jax.ShapeDtypeStruct((_B, _D), jnp.float32),
    mesh=plsc.VectorSubcoreMesh(core_axis_name="c", subcore_axis_name="s"),
    scratch_types=[
        pltpu.VMEM((_N_CHUNKS, _CHUNK), jnp.int32),
        pltpu.VMEM((_CHUNK, _D), jnp.float32),
        pltpu.SemaphoreType.DMA,
    ],
    compiler_params=pltpu.CompilerParams(use_tc_tiling_on_sc=False),
)
def _sc_gather(idx_hbm, table_hbm, out_hbm, idx_v, rows_v, sem):
    wid = lax.axis_index("s") * _NC + lax.axis_index("c")
    cbase = wid * _N_CHUNKS

    pltpu.sync_copy(idx_hbm.at[wid], idx_v)

    def _remap(j, carry):
        for i in range(_CHUNK // 16):
            v = idx_v[j, pl.ds(i * 16, 16)]
            u = jnp.minimum(plsc.bitcast(v, jnp.uint32), jnp.uint32(_NULL_IDX))
            idx_v[j, pl.ds(i * 16, 16)] = plsc.bitcast(u, jnp.int32)
        return carry

    lax.fori_loop(0, _N_CHUNKS, _remap, 0)

    def _chunk(j, carry):
        pltpu.async_copy(table_hbm.at[idx_v.at[j]], rows_v, sem).wait()
        pltpu.sync_copy(rows_v, out_hbm.at[pl.ds((cbase + j) * _CHUNK, _CHUNK)])
        return carry

    lax.fori_loop(0, _N_CHUNKS, _chunk, 0)


def kernel(blueprint_indices, table):
    idx3d = blueprint_indices.reshape(_NW, _N_CHUNKS, _CHUNK).astype(jnp.int32)
    out = _sc_gather(idx3d, table)
    return out.reshape(4096, 26, _D)


# speedup vs baseline: 1.1781x; 1.0744x over previous
import functools

import jax
import jax.numpy as jnp
from jax import lax
from jax.experimental import pallas as pl
from jax.experimental.pallas import tpu as pltpu
from jax.experimental.pallas import tpu_sc as plsc

_NUM_BLUEPRINTS = 100000
_NULL_IDX = _NUM_BLUEPRINTS
_D = 64
_NC = 2
_NS = 16
_NW = _NC * _NS
_B = 4096 * 26
_CHUNK = 128
_B_PER_W = _B // _NW
_N_CHUNKS = _B_PER_W // _CHUNK


_G = 2
_NGROUPS = _N_CHUNKS // _G


@functools.partial(
    pl.kernel,
    out_type=jax.ShapeDtypeStruct((_B, _D), jnp.float32),
    mesh=plsc.VectorSubcoreMesh(core_axis_name="c", subcore_axis_name="s"),
    scratch_types=[
        pltpu.VMEM((_N_CHUNKS, _CHUNK), jnp.int32),
        pltpu.VMEM((_G, _CHUNK, _D), jnp.float32),
        pltpu.VMEM((_G, _CHUNK, _D), jnp.float32),
        pltpu.SemaphoreType.DMA,
        pltpu.SemaphoreType.DMA,
        pltpu.SemaphoreType.DMA,
    ],
    compiler_params=pltpu.CompilerParams(use_tc_tiling_on_sc=False),
)
def _sc_gather(idx_hbm, table_hbm, out_hbm, idx_v, buf_a, buf_b, gsem, ssem_a,
               ssem_b):
    wid = lax.axis_index("s") * _NC + lax.axis_index("c")
    cbase = wid * _N_CHUNKS

    pltpu.sync_copy(idx_hbm.at[wid], idx_v)

    def _remap(j, carry):
        for i in range(_CHUNK // 16):
            v = idx_v[j, pl.ds(i * 16, 16)]
            u = jnp.minimum(plsc.bitcast(v, jnp.uint32), jnp.uint32(_NULL_IDX))
            idx_v[j, pl.ds(i * 16, 16)] = plsc.bitcast(u, jnp.int32)
        return carry

    lax.fori_loop(0, _N_CHUNKS, _remap, 0)

    def _out_slice(c):
        return out_hbm.at[pl.ds((cbase + c) * _CHUNK, _CHUNK)]

    def fire_gathers(g, buf):
        for b in range(_G):
            pltpu.async_copy(table_hbm.at[idx_v.at[g * _G + b]], buf.at[b],
                             gsem)

    def wait_gathers(g, buf):
        for b in range(_G):
            pltpu.make_async_copy(table_hbm.at[idx_v.at[g * _G + b]],
                                  buf.at[b], gsem).wait()

    def fire_stores(g, buf, ssem):
        for b in range(_G):
            pltpu.async_copy(buf.at[b], _out_slice(g * _G + b), ssem)

    def wait_stores(g, buf, ssem):
        for b in range(_G):
            pltpu.make_async_copy(buf.at[b], _out_slice(g * _G + b),
                                  ssem).wait()

    fire_gathers(0, buf_a)
    wait_gathers(0, buf_a)
    fire_stores(0, buf_a, ssem_a)
    fire_gathers(1, buf_b)

    def _pipe(gg, carry):
        g1 = 2 * gg + 1
        wait_gathers(g1, buf_b)
        fire_stores(g1, buf_b, ssem_b)
        wait_stores(g1 - 1, buf_a, ssem_a)
        fire_gathers(g1 + 1, buf_a)

        g2 = 2 * gg + 2
        wait_gathers(g2, buf_a)
        fire_stores(g2, buf_a, ssem_a)
        wait_stores(g2 - 1, buf_b, ssem_b)

        @pl.when(g2 < _NGROUPS - 1)
        def _():
            fire_gathers(g2 + 1, buf_b)

        return carry

    lax.fori_loop(0, (_NGROUPS - 1) // 2, _pipe, 0)
    wait_stores(_NGROUPS - 1, buf_a, ssem_a)


def kernel(blueprint_indices, table):
    idx3d = blueprint_indices.reshape(_NW, _N_CHUNKS, _CHUNK).astype(jnp.int32)
    out = _sc_gather(idx3d, table)
    return out.reshape(4096, 26, _D)
